# Initial kernel scaffold; baseline (speedup 1.0000x reference)
import jax
import jax.numpy as jnp
from jax.experimental import pallas as pl


def kernel(x, edge_index, etype, W_gcn, gt_weight, Wg, bg, W1, b1, W2, b2):
    raise NotImplementedError("write your pallas kernel here")


# speedup vs baseline: 1.0003x; 1.0003x over previous
import jax
import jax.numpy as jnp
from jax.experimental import pallas as pl

N = 10000
IN_DIM = 128
HID = 64
C = 2
L = 2
R = 4
NUM_CLASS = 16


def _head_body(X_ref, W1_ref, b1_ref, W2_ref, b2_ref, y_ref):
    X = X_ref[...]
    h = jnp.maximum(X @ W1_ref[...] + b1_ref[...], 0.0)
    y_ref[...] = h @ W2_ref[...] + b2_ref[...]


def kernel(x, edge_index, etype, W_gcn, gt_weight, Wg, bg, W1, b1, W2, b2):
    src = edge_index[0]
    dst = edge_index[1]
    H = [x @ W_gcn[c] for c in range(C)]
    for l in range(L):
        Filt = jax.nn.softmax(gt_weight[l], axis=-1)
        newH = []
        for c in range(C):
            ew = Filt[c][etype]
            self_w = Filt[c, R - 1]
            deg = jnp.zeros((N,), dtype=x.dtype).at[dst].add(ew) + self_w
            ew_n = ew / deg[dst]
            self_n = self_w / deg
            agg = jnp.zeros((N, HID), dtype=x.dtype).at[dst].add(
                ew_n[:, None] * H[c][src])
            agg = agg + self_n[:, None] * H[c]
            newH.append(jax.nn.relu(agg @ Wg + bg))
        H = newH
    X = jnp.concatenate(H, axis=1)
    y = pl.pallas_call(
        _head_body,
        out_shape=jax.ShapeDtypeStruct((N, NUM_CLASS), x.dtype),
    )(X, W1, b1.reshape(1, HID), W2, b2.reshape(1, NUM_CLASS))
    return y


# speedup vs baseline: 10.1853x; 10.1823x over previous
import functools

import jax
import jax.numpy as jnp
from jax import lax
from jax.experimental import pallas as pl
from jax.experimental.pallas import tpu as pltpu
from jax.experimental.pallas import tpu_sc as plsc

NC = 2
NS = 16
NW = NC * NS
ROW = 128
CHUNK_ROWS = 4


def _edge_mesh():
    return plsc.VectorSubcoreMesh(core_axis_name="c", subcore_axis_name="s")


def _make_edge_pass(acc_rows, hid, rows_pad):
    rows_per_worker = rows_pad // NW
    n_chunks = rows_per_worker // CHUNK_ROWS
    rows_per_sub = acc_rows // NS

    @functools.partial(
        pl.kernel,
        out_type=jax.ShapeDtypeStruct((NC, acc_rows, hid), jnp.float32),
        mesh=_edge_mesh(),
        scratch_types=[
            pltpu.VMEM((CHUNK_ROWS, ROW), jnp.int32),
            pltpu.VMEM((CHUNK_ROWS, ROW), jnp.int32),
            pltpu.VMEM((CHUNK_ROWS * ROW, hid), jnp.float32),
            pltpu.VMEM_SHARED((acc_rows, hid), jnp.float32),
            pltpu.SemaphoreType.DMA,
        ],
        compiler_params=pltpu.CompilerParams(use_tc_tiling_on_sc=False),
    )
    def edge_pass(h3_hbm, gidx_hbm, dstp_hbm, zeros_hbm, out_hbm,
                  gidx_v, dst_v, msg_v, acc_sh, sem):
        core = lax.axis_index("c")
        sub = lax.axis_index("s")
        wid = core * NS + sub

        my_acc = pl.ds(sub * rows_per_sub, rows_per_sub)
        pltpu.sync_copy(zeros_hbm.at[my_acc], acc_sh.at[my_acc])
        plsc.subcore_barrier()

        row0 = wid * rows_per_worker

        @pl.loop(0, n_chunks)
        def _chunk(i):
            r0 = row0 + i * CHUNK_ROWS
            pltpu.sync_copy(gidx_hbm.at[pl.ds(r0, CHUNK_ROWS)], gidx_v)
            pltpu.sync_copy(dstp_hbm.at[pl.ds(r0, CHUNK_ROWS)], dst_v)
            cps = [
                pltpu.async_copy(h3_hbm.at[gidx_v.at[j]],
                                 msg_v.at[pl.ds(j * ROW, ROW)], sem)
                for j in range(CHUNK_ROWS)
            ]
            for cp in cps:
                cp.wait()
            for j in range(CHUNK_ROWS):
                pltpu.sync_copy(msg_v.at[pl.ds(j * ROW, ROW)],
                                acc_sh.at[dst_v.at[j]], add=True)

        plsc.subcore_barrier()
        pltpu.sync_copy(acc_sh.at[my_acc], out_hbm.at[core].at[my_acc])

    return edge_pass


def _make_cnt_pass(cacc_rows, rows_pad):
    rows_per_worker = rows_pad // NW
    n_chunks = rows_per_worker // CHUNK_ROWS
    rows_per_sub = cacc_rows // NS

    @functools.partial(
        pl.kernel,
        out_type=jax.ShapeDtypeStruct((NC, cacc_rows, 16), jnp.float32),
        mesh=_edge_mesh(),
        scratch_types=[
            pltpu.VMEM((CHUNK_ROWS, ROW), jnp.int32),
            pltpu.VMEM((ROW, 16), jnp.float32),
            pltpu.VMEM_SHARED((cacc_rows, 16), jnp.float32),
        ],
        compiler_params=pltpu.CompilerParams(use_tc_tiling_on_sc=False),
    )
    def cnt_pass(cidx_hbm, ones_hbm, zeros_hbm, out_hbm,
                 cidx_v, ones_v, acc_sh):
        core = lax.axis_index("c")
        sub = lax.axis_index("s")
        wid = core * NS + sub

        my_acc = pl.ds(sub * rows_per_sub, rows_per_sub)
        pltpu.sync_copy(zeros_hbm.at[my_acc], acc_sh.at[my_acc])
        pltpu.sync_copy(ones_hbm, ones_v)
        plsc.subcore_barrier()

        row0 = wid * rows_per_worker

        @pl.loop(0, n_chunks)
        def _chunk(i):
            r0 = row0 + i * CHUNK_ROWS
            pltpu.sync_copy(cidx_hbm.at[pl.ds(r0, CHUNK_ROWS)], cidx_v)
            for j in range(CHUNK_ROWS):
                pltpu.sync_copy(ones_v, acc_sh.at[cidx_v.at[j]], add=True)

        plsc.subcore_barrier()
        pltpu.sync_copy(acc_sh.at[my_acc], out_hbm.at[core].at[my_acc])

    return cnt_pass


def _prep_body(etg_ref, src_ref, etc_ref, dstc_ref, gidx_ref, cidx_ref, *,
               n_nodes):
    gidx_ref[...] = etg_ref[...] * n_nodes + src_ref[...]
    cidx_ref[...] = etc_ref[...] * n_nodes + dstc_ref[...]


def _proj_body(x_ref, w_ref, out_ref):
    x = x_ref[...]
    for c in range(w_ref.shape[0]):
        out_ref[c] = jnp.dot(x, w_ref[c], preferred_element_type=jnp.float32)


def _make_scale_body(c, n_rel, n_nodes):
    def body(h_ref, gt_ref, o_ref):
        f = jax.nn.softmax(gt_ref[...], axis=-1)[c]
        h = h_ref[...]
        for r in range(n_rel):
            o_ref[pl.ds(r * n_nodes, n_nodes)] = f[r] * h
    return body


def _make_combine_body(c, n_rel):
    def body(p_ref, cnt_ref, h_ref, gt_ref, wg_ref, bg_ref, o_ref):
        f = jax.nn.softmax(gt_ref[...], axis=-1)[c]
        p = p_ref[...]
        s = p[0] + p[1]
        cp = cnt_ref[...]
        cnt = cp[0] + cp[1]
        deg = f[0] * cnt[0]
        for r in range(1, n_rel):
            deg = deg + f[r] * cnt[r]
        deg = deg + f[n_rel]
        h = h_ref[...]
        agg = (s + f[n_rel] * h) / deg
        o_ref[...] = jnp.maximum(
            jnp.dot(agg, wg_ref[...], preferred_element_type=jnp.float32)
            + bg_ref[...], 0.0)
    return body


def _head_body(x0_ref, x1_ref, w1_ref, b1_ref, w2_ref, b2_ref, y_ref):
    hid = x0_ref.shape[1]
    xw = (jnp.dot(x0_ref[...], w1_ref[pl.ds(0, hid)],
                  preferred_element_type=jnp.float32)
          + jnp.dot(x1_ref[...], w1_ref[pl.ds(hid, hid)],
                    preferred_element_type=jnp.float32))
    h = jnp.maximum(xw + b1_ref[...], 0.0)
    y_ref[...] = jnp.dot(h, w2_ref[...], preferred_element_type=jnp.float32) \
        + b2_ref[...]


def kernel(x, edge_index, etype, W_gcn, gt_weight, Wg, bg, W1, b1, W2, b2):
    n, in_dim = x.shape
    e = edge_index.shape[1]
    n_ch, _, hid = W_gcn.shape
    n_layer, _, n_rel_full = gt_weight.shape
    n_rel = n_rel_full - 1
    num_class = W2.shape[1]

    rows_e = e // ROW
    rows_pad = ((rows_e + NW * CHUNK_ROWS - 1)
                // (NW * CHUNK_ROWS)) * (NW * CHUNK_ROWS)
    pad_edges = rows_pad * ROW - e
    acc_rows = ((n + 16 + 127) // 128) * 128
    cacc_rows = ((n_rel * n + 16 + 127) // 128) * 128

    src = edge_index[0]
    dst = edge_index[1]
    padk = jnp.arange(pad_edges, dtype=jnp.int32) % 16
    zpad = jnp.zeros((pad_edges,), jnp.int32)
    et32 = etype.astype(jnp.int32)
    src_p = jnp.concatenate([src, zpad]).reshape(rows_pad, ROW)
    etg_p = jnp.concatenate([et32, zpad]).reshape(rows_pad, ROW)
    dsts_p = jnp.concatenate([dst, padk + n]).reshape(rows_pad, ROW)
    etc_p = jnp.concatenate(
        [et32, jnp.full((pad_edges,), n_rel, jnp.int32)]).reshape(rows_pad, ROW)
    dstc_p = jnp.concatenate([dst, padk]).reshape(rows_pad, ROW)
    zeros_acc = jnp.zeros((acc_rows, hid), jnp.float32)
    zeros_cnt = jnp.zeros((cacc_rows, 16), jnp.float32)
    ones_row = jnp.ones((ROW, 16), jnp.float32)

    gidx, cidx = pl.pallas_call(
        functools.partial(_prep_body, n_nodes=n),
        out_shape=[jax.ShapeDtypeStruct((rows_pad, ROW), jnp.int32),
                   jax.ShapeDtypeStruct((rows_pad, ROW), jnp.int32)],
    )(etg_p, src_p, etc_p, dstc_p)

    cnt_pass = _make_cnt_pass(cacc_rows, rows_pad)
    cnt_part = cnt_pass(cidx, ones_row, zeros_cnt)
    cnt3 = cnt_part[:, :n_rel * n, 0:1].reshape(NC, n_rel, n, 1)

    H0 = pl.pallas_call(
        _proj_body,
        out_shape=jax.ShapeDtypeStruct((n_ch, n, hid), jnp.float32),
    )(x, W_gcn)

    edge_pass = _make_edge_pass(acc_rows, hid, rows_pad)

    scale_calls = [
        pl.pallas_call(
            _make_scale_body(c, n_rel, n),
            out_shape=jax.ShapeDtypeStruct((n_rel * n, hid), jnp.float32),
        ) for c in range(n_ch)
    ]

    BLK = 2000
    grid = (n // BLK,)
    combine_calls = []
    for c in range(n_ch):
        combine_calls.append(pl.pallas_call(
            _make_combine_body(c, n_rel),
            grid=grid,
            in_specs=[
                pl.BlockSpec((NC, BLK, hid), lambda i: (0, i, 0)),
                pl.BlockSpec((NC, n_rel, BLK, 1), lambda i: (0, 0, i, 0)),
                pl.BlockSpec((BLK, hid), lambda i: (i, 0)),
                pl.BlockSpec((n_ch, n_rel_full), lambda i: (0, 0)),
                pl.BlockSpec((hid, hid), lambda i: (0, 0)),
                pl.BlockSpec((1, hid), lambda i: (0, 0)),
            ],
            out_specs=pl.BlockSpec((BLK, hid), lambda i: (i, 0)),
            out_shape=jax.ShapeDtypeStruct((n, hid), jnp.float32),
        ))

    H = [H0[c] for c in range(n_ch)]
    for l in range(n_layer):
        newH = []
        for c in range(n_ch):
            h3 = scale_calls[c](H[c], gt_weight[l])
            part = edge_pass(h3, gidx, dsts_p, zeros_acc)
            p2 = part[:, :n]
            newH.append(combine_calls[c](
                p2, cnt3, H[c], gt_weight[l], Wg, bg.reshape(1, hid)))
        H = newH

    y = pl.pallas_call(
        _head_body,
        out_shape=jax.ShapeDtypeStruct((n, num_class), jnp.float32),
    )(H[0], H[1], W1, b1.reshape(1, hid), W2, b2.reshape(1, num_class))
    return y


# speedup vs baseline: 11.3678x; 1.1161x over previous
import functools

import jax
import jax.numpy as jnp
from jax import lax
from jax.experimental import pallas as pl
from jax.experimental.pallas import tpu as pltpu
from jax.experimental.pallas import tpu_sc as plsc

NC = 2
NS = 16
NW = NC * NS
ROW = 128
CHUNK_ROWS = 2
CNT_CHUNK = 4


def _edge_mesh():
    return plsc.VectorSubcoreMesh(core_axis_name="c", subcore_axis_name="s")


def _make_edge_pass(acc_rows, hid, rows_pad):
    rows_per_worker = rows_pad // NW
    n_chunks = rows_per_worker // CHUNK_ROWS
    assert n_chunks % 2 == 0 and n_chunks >= 4
    rows_per_sub = acc_rows // NS
    cw = CHUNK_ROWS * ROW

    @functools.partial(
        pl.kernel,
        out_type=jax.ShapeDtypeStruct((NC, acc_rows, hid), jnp.float32),
        mesh=_edge_mesh(),
        scratch_types=[
            pltpu.VMEM((rows_per_worker, ROW), jnp.int32),
            pltpu.VMEM((rows_per_worker, ROW), jnp.int32),
            pltpu.VMEM((2, cw, hid), jnp.float32),
            pltpu.VMEM_SHARED((acc_rows, hid), jnp.float32),
            pltpu.SemaphoreType.DMA,
            pltpu.SemaphoreType.DMA,
            pltpu.SemaphoreType.DMA,
            pltpu.SemaphoreType.DMA,
            pltpu.SemaphoreType.DMA,
        ],
        compiler_params=pltpu.CompilerParams(use_tc_tiling_on_sc=False),
    )
    def edge_pass(h3_hbm, gidx_hbm, dstp_hbm, zeros_hbm, out_hbm,
                  gidx_all, dst_all, msg_v, acc_sh,
                  gsem0, gsem1, ssem0, ssem1, zsem):
        core = lax.axis_index("c")
        sub = lax.axis_index("s")
        wid = core * NS + sub
        gsem = (gsem0, gsem1)
        ssem = (ssem0, ssem1)

        my_acc = pl.ds(sub * rows_per_sub, rows_per_sub)
        zcp = pltpu.async_copy(zeros_hbm.at[my_acc], acc_sh.at[my_acc], zsem)
        row0 = wid * rows_per_worker
        pltpu.sync_copy(gidx_hbm.at[pl.ds(row0, rows_per_worker)], gidx_all)
        pltpu.sync_copy(dstp_hbm.at[pl.ds(row0, rows_per_worker)], dst_all)
        zcp.wait()
        plsc.subcore_barrier()

        def issue_gathers(i, s):
            for j in range(CHUNK_ROWS):
                pltpu.async_copy(h3_hbm.at[gidx_all.at[i * CHUNK_ROWS + j]],
                                 msg_v.at[s].at[pl.ds(j * ROW, ROW)], gsem[s])

        def issue_scatters(i, s):
            for j in range(CHUNK_ROWS):
                pltpu.async_copy(msg_v.at[s].at[pl.ds(j * ROW, ROW)],
                                 acc_sh.at[dst_all.at[i * CHUNK_ROWS + j]],
                                 ssem[s], add=True)

        def drain(sem, s):
            pltpu.make_async_copy(h3_hbm.at[pl.ds(0, cw)], msg_v.at[s],
                                  sem).wait()

        issue_gathers(0, 0)
        issue_gathers(1, 1)

        @pl.loop(0, (n_chunks - 2) // 2)
        def _pair(k):
            i0 = 2 * k
            drain(gsem[0], 0)
            issue_scatters(i0, 0)
            drain(ssem[0], 0)
            issue_gathers(i0 + 2, 0)
            drain(gsem[1], 1)
            issue_scatters(i0 + 1, 1)
            drain(ssem[1], 1)
            issue_gathers(i0 + 3, 1)

        drain(gsem[0], 0)
        issue_scatters(n_chunks - 2, 0)
        drain(gsem[1], 1)
        issue_scatters(n_chunks - 1, 1)
        drain(ssem[0], 0)
        drain(ssem[1], 1)

        plsc.subcore_barrier()
        pltpu.sync_copy(acc_sh.at[my_acc], out_hbm.at[core].at[my_acc])

    return edge_pass


def _make_cnt_pass(cacc_rows, rows_pad):
    rows_per_worker = rows_pad // NW
    n_chunks = rows_per_worker // CNT_CHUNK
    rows_per_sub = cacc_rows // NS

    @functools.partial(
        pl.kernel,
        out_type=jax.ShapeDtypeStruct((NC, cacc_rows, 16), jnp.float32),
        mesh=_edge_mesh(),
        scratch_types=[
            pltpu.VMEM((CNT_CHUNK, ROW), jnp.int32),
            pltpu.VMEM((ROW, 16), jnp.float32),
            pltpu.VMEM_SHARED((cacc_rows, 16), jnp.float32),
        ],
        compiler_params=pltpu.CompilerParams(use_tc_tiling_on_sc=False),
    )
    def cnt_pass(cidx_hbm, ones_hbm, zeros_hbm, out_hbm,
                 cidx_v, ones_v, acc_sh):
        core = lax.axis_index("c")
        sub = lax.axis_index("s")
        wid = core * NS + sub

        my_acc = pl.ds(sub * rows_per_sub, rows_per_sub)
        pltpu.sync_copy(zeros_hbm.at[my_acc], acc_sh.at[my_acc])
        pltpu.sync_copy(ones_hbm, ones_v)
        plsc.subcore_barrier()

        row0 = wid * rows_per_worker

        @pl.loop(0, n_chunks)
        def _chunk(i):
            r0 = row0 + i * CNT_CHUNK
            pltpu.sync_copy(cidx_hbm.at[pl.ds(r0, CNT_CHUNK)], cidx_v)
            for j in range(CNT_CHUNK):
                pltpu.sync_copy(ones_v, acc_sh.at[cidx_v.at[j]], add=True)

        plsc.subcore_barrier()
        pltpu.sync_copy(acc_sh.at[my_acc], out_hbm.at[core].at[my_acc])

    return cnt_pass


def _prep_body(etg_ref, src_ref, etc_ref, dstc_ref, gidx_ref, cidx_ref, *,
               n_nodes):
    gidx_ref[...] = etg_ref[...] * n_nodes + src_ref[...]
    cidx_ref[...] = etc_ref[...] * n_nodes + dstc_ref[...]


def _proj_body(x_ref, w_ref, out_ref):
    x = x_ref[...]
    for c in range(w_ref.shape[0]):
        out_ref[c] = jnp.dot(x, w_ref[c], preferred_element_type=jnp.float32)


def _make_scale_body(c, n_rel, n_nodes):
    def body(h_ref, gt_ref, o_ref):
        f = jax.nn.softmax(gt_ref[...], axis=-1)[c]
        h = h_ref[...]
        for r in range(n_rel):
            o_ref[pl.ds(r * n_nodes, n_nodes)] = f[r] * h
    return body


def _make_combine_body(c, n_rel):
    def body(p_ref, cnt_ref, h_ref, gt_ref, wg_ref, bg_ref, o_ref):
        f = jax.nn.softmax(gt_ref[...], axis=-1)[c]
        p = p_ref[...]
        s = p[0] + p[1]
        cp = cnt_ref[...]
        cnt = cp[0] + cp[1]
        deg = f[0] * cnt[0]
        for r in range(1, n_rel):
            deg = deg + f[r] * cnt[r]
        deg = deg + f[n_rel]
        h = h_ref[...]
        agg = (s + f[n_rel] * h) / deg
        o_ref[...] = jnp.maximum(
            jnp.dot(agg, wg_ref[...], preferred_element_type=jnp.float32)
            + bg_ref[...], 0.0)
    return body


def _head_body(x0_ref, x1_ref, w1_ref, b1_ref, w2_ref, b2_ref, y_ref):
    hid = x0_ref.shape[1]
    xw = (jnp.dot(x0_ref[...], w1_ref[pl.ds(0, hid)],
                  preferred_element_type=jnp.float32)
          + jnp.dot(x1_ref[...], w1_ref[pl.ds(hid, hid)],
                    preferred_element_type=jnp.float32))
    h = jnp.maximum(xw + b1_ref[...], 0.0)
    y_ref[...] = jnp.dot(h, w2_ref[...], preferred_element_type=jnp.float32) \
        + b2_ref[...]


def kernel(x, edge_index, etype, W_gcn, gt_weight, Wg, bg, W1, b1, W2, b2):
    n, in_dim = x.shape
    e = edge_index.shape[1]
    n_ch, _, hid = W_gcn.shape
    n_layer, _, n_rel_full = gt_weight.shape
    n_rel = n_rel_full - 1
    num_class = W2.shape[1]

    rows_e = e // ROW
    import math
    row_gran = NW * math.lcm(CHUNK_ROWS * 2, CNT_CHUNK)
    rows_pad = ((rows_e + row_gran - 1) // row_gran) * row_gran
    pad_edges = rows_pad * ROW - e
    acc_rows = ((n + 16 + 127) // 128) * 128
    cacc_rows = ((n_rel * n + 16 + 127) // 128) * 128

    src = edge_index[0]
    dst = edge_index[1]
    padk = jnp.arange(pad_edges, dtype=jnp.int32) % 16
    zpad = jnp.zeros((pad_edges,), jnp.int32)
    et32 = etype.astype(jnp.int32)
    src_p = jnp.concatenate([src, zpad]).reshape(rows_pad, ROW)
    etg_p = jnp.concatenate([et32, zpad]).reshape(rows_pad, ROW)
    dsts_p = jnp.concatenate([dst, padk + n]).reshape(rows_pad, ROW)
    etc_p = jnp.concatenate(
        [et32, jnp.full((pad_edges,), n_rel, jnp.int32)]).reshape(rows_pad, ROW)
    dstc_p = jnp.concatenate([dst, padk]).reshape(rows_pad, ROW)
    zeros_acc = jnp.zeros((acc_rows, hid), jnp.float32)
    zeros_cnt = jnp.zeros((cacc_rows, 16), jnp.float32)
    ones_row = jnp.ones((ROW, 16), jnp.float32)

    gidx, cidx = pl.pallas_call(
        functools.partial(_prep_body, n_nodes=n),
        out_shape=[jax.ShapeDtypeStruct((rows_pad, ROW), jnp.int32),
                   jax.ShapeDtypeStruct((rows_pad, ROW), jnp.int32)],
    )(etg_p, src_p, etc_p, dstc_p)

    cnt_pass = _make_cnt_pass(cacc_rows, rows_pad)
    cnt_part = cnt_pass(cidx, ones_row, zeros_cnt)
    cnt3 = cnt_part[:, :n_rel * n, 0:1].reshape(NC, n_rel, n, 1)

    H0 = pl.pallas_call(
        _proj_body,
        out_shape=jax.ShapeDtypeStruct((n_ch, n, hid), jnp.float32),
    )(x, W_gcn)

    edge_pass = _make_edge_pass(acc_rows, hid, rows_pad)

    scale_calls = [
        pl.pallas_call(
            _make_scale_body(c, n_rel, n),
            out_shape=jax.ShapeDtypeStruct((n_rel * n, hid), jnp.float32),
        ) for c in range(n_ch)
    ]

    BLK = 2000
    grid = (n // BLK,)
    combine_calls = []
    for c in range(n_ch):
        combine_calls.append(pl.pallas_call(
            _make_combine_body(c, n_rel),
            grid=grid,
            in_specs=[
                pl.BlockSpec((NC, BLK, hid), lambda i: (0, i, 0)),
                pl.BlockSpec((NC, n_rel, BLK, 1), lambda i: (0, 0, i, 0)),
                pl.BlockSpec((BLK, hid), lambda i: (i, 0)),
                pl.BlockSpec((n_ch, n_rel_full), lambda i: (0, 0)),
                pl.BlockSpec((hid, hid), lambda i: (0, 0)),
                pl.BlockSpec((1, hid), lambda i: (0, 0)),
            ],
            out_specs=pl.BlockSpec((BLK, hid), lambda i: (i, 0)),
            out_shape=jax.ShapeDtypeStruct((n, hid), jnp.float32),
        ))

    H = [H0[c] for c in range(n_ch)]
    for l in range(n_layer):
        newH = []
        for c in range(n_ch):
            h3 = scale_calls[c](H[c], gt_weight[l])
            part = edge_pass(h3, gidx, dsts_p, zeros_acc)
            p2 = part[:, :n]
            newH.append(combine_calls[c](
                p2, cnt3, H[c], gt_weight[l], Wg, bg.reshape(1, hid)))
        H = newH

    y = pl.pallas_call(
        _head_body,
        out_shape=jax.ShapeDtypeStruct((n, num_class), jnp.float32),
    )(H[0], H[1], W1, b1.reshape(1, hid), W2, b2.reshape(1, num_class))
    return y


# speedup vs baseline: 12.1812x; 1.0715x over previous
import functools

import jax
import jax.numpy as jnp
from jax import lax
from jax.experimental import pallas as pl
from jax.experimental.pallas import tpu as pltpu
from jax.experimental.pallas import tpu_sc as plsc

NC = 2
NS = 16
NW = NC * NS
ROW = 128
CHUNK_ROWS = 2
CNT_CHUNK = 4


def _edge_mesh():
    return plsc.VectorSubcoreMesh(core_axis_name="c", subcore_axis_name="s")


def _make_edge_pass(acc_rows, hid, rows_pad):
    rows_per_pair = rows_pad // NS
    r0_rows = (int(rows_per_pair * 0.775) // (2 * CHUNK_ROWS)) * 2 * CHUNK_ROWS
    r1_rows = rows_per_pair - r0_rows
    assert r1_rows % (2 * CHUNK_ROWS) == 0 and r0_rows >= 4 and r1_rows >= 4
    rows_per_sub = acc_rows // NS
    cw = CHUNK_ROWS * ROW

    @functools.partial(
        pl.kernel,
        out_type=jax.ShapeDtypeStruct((NC, acc_rows, hid), jnp.float32),
        mesh=_edge_mesh(),
        scratch_types=[
            pltpu.VMEM((r0_rows, ROW), jnp.int32),
            pltpu.VMEM((r0_rows, ROW), jnp.int32),
            pltpu.VMEM((2, cw, hid), jnp.float32),
            pltpu.VMEM_SHARED((acc_rows, hid), jnp.float32),
            pltpu.SemaphoreType.DMA,
            pltpu.SemaphoreType.DMA,
            pltpu.SemaphoreType.DMA,
            pltpu.SemaphoreType.DMA,
            pltpu.SemaphoreType.DMA,
        ],
        compiler_params=pltpu.CompilerParams(use_tc_tiling_on_sc=False),
    )
    def edge_pass(h3_hbm, gidx_hbm, dstp_hbm, zeros_hbm, out_hbm,
                  gidx_all, dst_all, msg_v, acc_sh,
                  gsem0, gsem1, ssem0, ssem1, zsem):
        core = lax.axis_index("c")
        sub = lax.axis_index("s")
        wid = core * NS + sub
        gsem = (gsem0, gsem1)
        ssem = (ssem0, ssem1)

        my_acc = pl.ds(sub * rows_per_sub, rows_per_sub)
        zcp = pltpu.async_copy(zeros_hbm.at[my_acc], acc_sh.at[my_acc], zsem)
        zcp.wait()
        plsc.subcore_barrier()

        def issue_gathers(i, s):
            for j in range(CHUNK_ROWS):
                pltpu.async_copy(h3_hbm.at[gidx_all.at[i * CHUNK_ROWS + j]],
                                 msg_v.at[s].at[pl.ds(j * ROW, ROW)], gsem[s])

        def issue_scatters(i, s):
            for j in range(CHUNK_ROWS):
                pltpu.async_copy(msg_v.at[s].at[pl.ds(j * ROW, ROW)],
                                 acc_sh.at[dst_all.at[i * CHUNK_ROWS + j]],
                                 ssem[s], add=True)

        def drain(sem, s):
            pltpu.make_async_copy(h3_hbm.at[pl.ds(0, cw)], msg_v.at[s],
                                  sem).wait()

        def run_worker(row0, rows):
            n_chunks = rows // CHUNK_ROWS
            pltpu.sync_copy(gidx_hbm.at[pl.ds(row0, rows)],
                            gidx_all.at[pl.ds(0, rows)])
            pltpu.sync_copy(dstp_hbm.at[pl.ds(row0, rows)],
                            dst_all.at[pl.ds(0, rows)])
            issue_gathers(0, 0)
            issue_gathers(1, 1)

            @pl.loop(0, (n_chunks - 2) // 2)
            def _pair(k):
                i0 = 2 * k
                drain(gsem[0], 0)
                issue_scatters(i0, 0)
                drain(ssem[0], 0)
                issue_gathers(i0 + 2, 0)
                drain(gsem[1], 1)
                issue_scatters(i0 + 1, 1)
                drain(ssem[1], 1)
                issue_gathers(i0 + 3, 1)

            drain(gsem[0], 0)
            issue_scatters(n_chunks - 2, 0)
            drain(gsem[1], 1)
            issue_scatters(n_chunks - 1, 1)
            drain(ssem[0], 0)
            drain(ssem[1], 1)

        @pl.when(core == 0)
        def _sc0():
            run_worker(sub * r0_rows, r0_rows)

        @pl.when(core == 1)
        def _sc1():
            run_worker(NS * r0_rows + sub * r1_rows, r1_rows)

        plsc.subcore_barrier()
        pltpu.sync_copy(acc_sh.at[my_acc], out_hbm.at[core].at[my_acc])

    return edge_pass


def _make_cnt_pass(cacc_rows, rows_pad):
    rows_per_worker = rows_pad // NW
    n_chunks = rows_per_worker // CNT_CHUNK
    rows_per_sub = cacc_rows // NS

    @functools.partial(
        pl.kernel,
        out_type=jax.ShapeDtypeStruct((NC, cacc_rows, 16), jnp.float32),
        mesh=_edge_mesh(),
        scratch_types=[
            pltpu.VMEM((CNT_CHUNK, ROW), jnp.int32),
            pltpu.VMEM((ROW, 16), jnp.float32),
            pltpu.VMEM_SHARED((cacc_rows, 16), jnp.float32),
        ],
        compiler_params=pltpu.CompilerParams(use_tc_tiling_on_sc=False),
    )
    def cnt_pass(cidx_hbm, ones_hbm, zeros_hbm, out_hbm,
                 cidx_v, ones_v, acc_sh):
        core = lax.axis_index("c")
        sub = lax.axis_index("s")
        wid = core * NS + sub

        my_acc = pl.ds(sub * rows_per_sub, rows_per_sub)
        pltpu.sync_copy(zeros_hbm.at[my_acc], acc_sh.at[my_acc])
        pltpu.sync_copy(ones_hbm, ones_v)
        plsc.subcore_barrier()

        row0 = wid * rows_per_worker

        @pl.loop(0, n_chunks)
        def _chunk(i):
            r0 = row0 + i * CNT_CHUNK
            pltpu.sync_copy(cidx_hbm.at[pl.ds(r0, CNT_CHUNK)], cidx_v)
            for j in range(CNT_CHUNK):
                pltpu.sync_copy(ones_v, acc_sh.at[cidx_v.at[j]], add=True)

        plsc.subcore_barrier()
        pltpu.sync_copy(acc_sh.at[my_acc], out_hbm.at[core].at[my_acc])

    return cnt_pass


def _prep_body(etg_ref, src_ref, etc_ref, dstc_ref, gidx_ref, cidx_ref, *,
               n_nodes):
    gidx_ref[...] = etg_ref[...] * n_nodes + src_ref[...]
    cidx_ref[...] = etc_ref[...] * n_nodes + dstc_ref[...]


def _proj_body(x_ref, w_ref, out_ref):
    x = x_ref[...]
    for c in range(w_ref.shape[0]):
        out_ref[c] = jnp.dot(x, w_ref[c], preferred_element_type=jnp.float32)


def _make_scale_body(c, n_rel, n_nodes):
    def body(h_ref, gt_ref, o_ref):
        f = jax.nn.softmax(gt_ref[...], axis=-1)[c]
        h = h_ref[...]
        for r in range(n_rel):
            o_ref[pl.ds(r * n_nodes, n_nodes)] = f[r] * h
    return body


def _make_combine_body(c, n_rel):
    def body(p_ref, c0_ref, c1_ref, c2_ref, h_ref, gt_ref, wg_ref, bg_ref,
             o_ref):
        f = jax.nn.softmax(gt_ref[...], axis=-1)[c]
        p = p_ref[...]
        s = p[0] + p[1]
        c16 = (f[0] * c0_ref[...] + f[1] * c1_ref[...]
               + f[2] * c2_ref[...])
        deg = c16[0, :, 0:1] + c16[1, :, 0:1] + f[n_rel]
        h = h_ref[...]
        agg = (s + f[n_rel] * h) / deg
        o_ref[...] = jnp.maximum(
            jnp.dot(agg, wg_ref[...], preferred_element_type=jnp.float32)
            + bg_ref[...], 0.0)
    return body


def _head_body(x0_ref, x1_ref, w1_ref, b1_ref, w2_ref, b2_ref, y_ref):
    hid = x0_ref.shape[1]
    xw = (jnp.dot(x0_ref[...], w1_ref[pl.ds(0, hid)],
                  preferred_element_type=jnp.float32)
          + jnp.dot(x1_ref[...], w1_ref[pl.ds(hid, hid)],
                    preferred_element_type=jnp.float32))
    h = jnp.maximum(xw + b1_ref[...], 0.0)
    y_ref[...] = jnp.dot(h, w2_ref[...], preferred_element_type=jnp.float32) \
        + b2_ref[...]


def kernel(x, edge_index, etype, W_gcn, gt_weight, Wg, bg, W1, b1, W2, b2):
    n, in_dim = x.shape
    e = edge_index.shape[1]
    n_ch, _, hid = W_gcn.shape
    n_layer, _, n_rel_full = gt_weight.shape
    n_rel = n_rel_full - 1
    num_class = W2.shape[1]

    rows_e = e // ROW
    import math
    row_gran = NW * math.lcm(CHUNK_ROWS * 2, CNT_CHUNK)
    rows_pad = ((rows_e + row_gran - 1) // row_gran) * row_gran
    pad_edges = rows_pad * ROW - e
    acc_rows = ((n + 16 + 127) // 128) * 128
    cacc_rows = ((n_rel * n + 16 + 127) // 128) * 128

    src = edge_index[0]
    dst = edge_index[1]
    padk = jnp.arange(pad_edges, dtype=jnp.int32) % 16
    zpad = jnp.zeros((pad_edges,), jnp.int32)
    et32 = etype.astype(jnp.int32)
    src_p = jnp.concatenate([src, zpad]).reshape(rows_pad, ROW)
    etg_p = jnp.concatenate([et32, zpad]).reshape(rows_pad, ROW)
    dsts_p = jnp.concatenate([dst, padk + n]).reshape(rows_pad, ROW)
    etc_p = jnp.concatenate(
        [et32, jnp.full((pad_edges,), n_rel, jnp.int32)]).reshape(rows_pad, ROW)
    dstc_p = jnp.concatenate([dst, padk]).reshape(rows_pad, ROW)
    zeros_acc = jnp.zeros((acc_rows, hid), jnp.float32)
    zeros_cnt = jnp.zeros((cacc_rows, 16), jnp.float32)
    ones_row = jnp.ones((ROW, 16), jnp.float32)

    gidx, cidx = pl.pallas_call(
        functools.partial(_prep_body, n_nodes=n),
        out_shape=[jax.ShapeDtypeStruct((rows_pad, ROW), jnp.int32),
                   jax.ShapeDtypeStruct((rows_pad, ROW), jnp.int32)],
    )(etg_p, src_p, etc_p, dstc_p)

    cnt_pass = _make_cnt_pass(cacc_rows, rows_pad)
    cnt_part = cnt_pass(cidx, ones_row, zeros_cnt)

    H0 = pl.pallas_call(
        _proj_body,
        out_shape=jax.ShapeDtypeStruct((n_ch, n, hid), jnp.float32),
    )(x, W_gcn)

    edge_pass = _make_edge_pass(acc_rows, hid, rows_pad)

    scale_calls = [
        pl.pallas_call(
            _make_scale_body(c, n_rel, n),
            out_shape=jax.ShapeDtypeStruct((n_rel * n, hid), jnp.float32),
        ) for c in range(n_ch)
    ]

    BLK = 2000
    assert n % BLK == 0 and n_rel == 3
    nb = n // BLK
    grid = (nb,)
    combine_calls = []
    for c in range(n_ch):
        combine_calls.append(pl.pallas_call(
            _make_combine_body(c, n_rel),
            grid=grid,
            in_specs=[
                pl.BlockSpec((NC, BLK, hid), lambda i: (0, i, 0)),
                pl.BlockSpec((NC, BLK, 16), lambda i: (0, i, 0)),
                pl.BlockSpec((NC, BLK, 16), lambda i, _nb=nb: (0, _nb + i, 0)),
                pl.BlockSpec((NC, BLK, 16),
                             lambda i, _nb=nb: (0, 2 * _nb + i, 0)),
                pl.BlockSpec((BLK, hid), lambda i: (i, 0)),
                pl.BlockSpec((n_ch, n_rel_full), lambda i: (0, 0)),
                pl.BlockSpec((hid, hid), lambda i: (0, 0)),
                pl.BlockSpec((1, hid), lambda i: (0, 0)),
            ],
            out_specs=pl.BlockSpec((BLK, hid), lambda i: (i, 0)),
            out_shape=jax.ShapeDtypeStruct((n, hid), jnp.float32),
        ))

    H = [H0[c] for c in range(n_ch)]
    for l in range(n_layer):
        newH = []
        for c in range(n_ch):
            h3 = scale_calls[c](H[c], gt_weight[l])
            part = edge_pass(h3, gidx, dsts_p, zeros_acc)
            newH.append(combine_calls[c](
                part, cnt_part, cnt_part, cnt_part, H[c], gt_weight[l],
                Wg, bg.reshape(1, hid)))
        H = newH

    y = pl.pallas_call(
        _head_body,
        out_shape=jax.ShapeDtypeStruct((n, num_class), jnp.float32),
    )(H[0], H[1], W1, b1.reshape(1, hid), W2, b2.reshape(1, num_class))
    return y


# speedup vs baseline: 12.1923x; 1.0009x over previous
import functools

import jax
import jax.numpy as jnp
from jax import lax
from jax.experimental import pallas as pl
from jax.experimental.pallas import tpu as pltpu
from jax.experimental.pallas import tpu_sc as plsc

NC = 2
NS = 16
NW = NC * NS
ROW = 128
CHUNK_ROWS = 2
CNT_CHUNK = 4


def _edge_mesh():
    return plsc.VectorSubcoreMesh(core_axis_name="c", subcore_axis_name="s")


def _make_edge_pass(acc_rows, hid, rows_pad):
    rows_per_pair = rows_pad // NS
    r0_rows = (int(rows_per_pair * 0.775) // (2 * CHUNK_ROWS)) * 2 * CHUNK_ROWS
    r1_rows = rows_per_pair - r0_rows
    assert r1_rows % (2 * CHUNK_ROWS) == 0 and r0_rows >= 4 and r1_rows >= 4
    rows_per_sub = acc_rows // NS
    cw = CHUNK_ROWS * ROW

    @functools.partial(
        pl.kernel,
        out_type=jax.ShapeDtypeStruct((NC, acc_rows, hid), jnp.float32),
        mesh=_edge_mesh(),
        scratch_types=[
            pltpu.VMEM((r0_rows, ROW), jnp.int32),
            pltpu.VMEM((r0_rows, ROW), jnp.int32),
            pltpu.VMEM((2, cw, hid), jnp.float32),
            pltpu.VMEM_SHARED((acc_rows, hid), jnp.float32),
            pltpu.SemaphoreType.DMA,
            pltpu.SemaphoreType.DMA,
            pltpu.SemaphoreType.DMA,
            pltpu.SemaphoreType.DMA,
            pltpu.SemaphoreType.DMA,
        ],
        compiler_params=pltpu.CompilerParams(use_tc_tiling_on_sc=False),
    )
    def edge_pass(h3_hbm, gidx_hbm, dstp_hbm, zeros_hbm, out_hbm,
                  gidx_all, dst_all, msg_v, acc_sh,
                  gsem0, gsem1, ssem0, ssem1, zsem):
        core = lax.axis_index("c")
        sub = lax.axis_index("s")
        wid = core * NS + sub
        gsem = (gsem0, gsem1)
        ssem = (ssem0, ssem1)

        my_acc = pl.ds(sub * rows_per_sub, rows_per_sub)
        with jax.named_scope("zinit"):
            zcp = pltpu.async_copy(zeros_hbm.at[my_acc], acc_sh.at[my_acc],
                                   zsem)
            zcp.wait()
            plsc.subcore_barrier()

        def issue_gathers(i, s):
            for j in range(CHUNK_ROWS):
                pltpu.async_copy(h3_hbm.at[gidx_all.at[i * CHUNK_ROWS + j]],
                                 msg_v.at[s].at[pl.ds(j * ROW, ROW)], gsem[s])

        def issue_scatters(i, s):
            for j in range(CHUNK_ROWS):
                pltpu.async_copy(msg_v.at[s].at[pl.ds(j * ROW, ROW)],
                                 acc_sh.at[dst_all.at[i * CHUNK_ROWS + j]],
                                 ssem[s], add=True)

        def drain(sem, s):
            pltpu.make_async_copy(h3_hbm.at[pl.ds(0, cw)], msg_v.at[s],
                                  sem).wait()

        def run_worker(row0, rows):
            n_chunks = rows // CHUNK_ROWS
            pltpu.sync_copy(gidx_hbm.at[pl.ds(row0, rows)],
                            gidx_all.at[pl.ds(0, rows)])
            pltpu.sync_copy(dstp_hbm.at[pl.ds(row0, rows)],
                            dst_all.at[pl.ds(0, rows)])
            issue_gathers(0, 0)
            issue_gathers(1, 1)

            @pl.loop(0, (n_chunks - 2) // 2)
            def _pair(k):
                i0 = 2 * k
                drain(gsem[0], 0)
                issue_scatters(i0, 0)
                drain(ssem[0], 0)
                issue_gathers(i0 + 2, 0)
                drain(gsem[1], 1)
                issue_scatters(i0 + 1, 1)
                drain(ssem[1], 1)
                issue_gathers(i0 + 3, 1)

            drain(gsem[0], 0)
            issue_scatters(n_chunks - 2, 0)
            drain(gsem[1], 1)
            issue_scatters(n_chunks - 1, 1)
            drain(ssem[0], 0)
            drain(ssem[1], 1)

        with jax.named_scope("edges"):
            @pl.when(core == 0)
            def _sc0():
                run_worker(sub * r0_rows, r0_rows)

            @pl.when(core == 1)
            def _sc1():
                run_worker(NS * r0_rows + sub * r1_rows, r1_rows)

            plsc.subcore_barrier()
        with jax.named_scope("flush"):
            pltpu.sync_copy(acc_sh.at[my_acc], out_hbm.at[core].at[my_acc])

    return edge_pass


def _make_cnt_pass(cacc_rows, rows_pad):
    rows_per_worker = rows_pad // NW
    n_chunks = rows_per_worker // CNT_CHUNK
    rows_per_sub = cacc_rows // NS

    @functools.partial(
        pl.kernel,
        out_type=jax.ShapeDtypeStruct((NC, cacc_rows, 16), jnp.float32),
        mesh=_edge_mesh(),
        scratch_types=[
            pltpu.VMEM((CNT_CHUNK, ROW), jnp.int32),
            pltpu.VMEM((ROW, 16), jnp.float32),
            pltpu.VMEM_SHARED((cacc_rows, 16), jnp.float32),
        ],
        compiler_params=pltpu.CompilerParams(use_tc_tiling_on_sc=False),
    )
    def cnt_pass(cidx_hbm, ones_hbm, zeros_hbm, out_hbm,
                 cidx_v, ones_v, acc_sh):
        core = lax.axis_index("c")
        sub = lax.axis_index("s")
        wid = core * NS + sub

        my_acc = pl.ds(sub * rows_per_sub, rows_per_sub)
        pltpu.sync_copy(zeros_hbm.at[my_acc], acc_sh.at[my_acc])
        pltpu.sync_copy(ones_hbm, ones_v)
        plsc.subcore_barrier()

        row0 = wid * rows_per_worker

        @pl.loop(0, n_chunks)
        def _chunk(i):
            r0 = row0 + i * CNT_CHUNK
            pltpu.sync_copy(cidx_hbm.at[pl.ds(r0, CNT_CHUNK)], cidx_v)
            for j in range(CNT_CHUNK):
                pltpu.sync_copy(ones_v, acc_sh.at[cidx_v.at[j]], add=True)

        plsc.subcore_barrier()
        pltpu.sync_copy(acc_sh.at[my_acc], out_hbm.at[core].at[my_acc])

    return cnt_pass


def _prep_body(etg_ref, src_ref, etc_ref, dstc_ref, gidx_ref, cidx_ref, *,
               n_nodes):
    gidx_ref[...] = etg_ref[...] * n_nodes + src_ref[...]
    cidx_ref[...] = etc_ref[...] * n_nodes + dstc_ref[...]


def _proj_body(x_ref, w_ref, out_ref):
    x = x_ref[...]
    for c in range(w_ref.shape[0]):
        out_ref[c] = jnp.dot(x, w_ref[c], preferred_element_type=jnp.float32)


def _make_scale_body(c, n_rel, n_nodes):
    def body(h_ref, gt_ref, o_ref):
        f = jax.nn.softmax(gt_ref[...], axis=-1)[c]
        h = h_ref[...]
        for r in range(n_rel):
            o_ref[pl.ds(r * n_nodes, n_nodes)] = f[r] * h
    return body


def _make_combine_body(c, n_rel):
    def body(p_ref, c0_ref, c1_ref, c2_ref, h_ref, gt_ref, wg_ref, bg_ref,
             o_ref):
        f = jax.nn.softmax(gt_ref[...], axis=-1)[c]
        p = p_ref[...]
        s = p[0] + p[1]
        c16 = (f[0] * c0_ref[...] + f[1] * c1_ref[...]
               + f[2] * c2_ref[...])
        deg = c16[0, :, 0:1] + c16[1, :, 0:1] + f[n_rel]
        h = h_ref[...]
        agg = (s + f[n_rel] * h) / deg
        o_ref[...] = jnp.maximum(
            jnp.dot(agg, wg_ref[...], preferred_element_type=jnp.float32)
            + bg_ref[...], 0.0)
    return body


def _head_body(x0_ref, x1_ref, w1_ref, b1_ref, w2_ref, b2_ref, y_ref):
    hid = x0_ref.shape[1]
    xw = (jnp.dot(x0_ref[...], w1_ref[pl.ds(0, hid)],
                  preferred_element_type=jnp.float32)
          + jnp.dot(x1_ref[...], w1_ref[pl.ds(hid, hid)],
                    preferred_element_type=jnp.float32))
    h = jnp.maximum(xw + b1_ref[...], 0.0)
    y_ref[...] = jnp.dot(h, w2_ref[...], preferred_element_type=jnp.float32) \
        + b2_ref[...]


def kernel(x, edge_index, etype, W_gcn, gt_weight, Wg, bg, W1, b1, W2, b2):
    n, in_dim = x.shape
    e = edge_index.shape[1]
    n_ch, _, hid = W_gcn.shape
    n_layer, _, n_rel_full = gt_weight.shape
    n_rel = n_rel_full - 1
    num_class = W2.shape[1]

    rows_e = e // ROW
    import math
    row_gran = NW * math.lcm(CHUNK_ROWS * 2, CNT_CHUNK)
    rows_pad = ((rows_e + row_gran - 1) // row_gran) * row_gran
    pad_edges = rows_pad * ROW - e
    acc_rows = ((n + 16 + 127) // 128) * 128
    cacc_rows = ((n_rel * n + 16 + 127) // 128) * 128

    src = edge_index[0]
    dst = edge_index[1]
    padk = jnp.arange(pad_edges, dtype=jnp.int32) % 16
    zpad = jnp.zeros((pad_edges,), jnp.int32)
    et32 = etype.astype(jnp.int32)
    src_p = jnp.concatenate([src, zpad]).reshape(rows_pad, ROW)
    etg_p = jnp.concatenate([et32, zpad]).reshape(rows_pad, ROW)
    dsts_p = jnp.concatenate([dst, padk + n]).reshape(rows_pad, ROW)
    etc_p = jnp.concatenate(
        [et32, jnp.full((pad_edges,), n_rel, jnp.int32)]).reshape(rows_pad, ROW)
    dstc_p = jnp.concatenate([dst, padk]).reshape(rows_pad, ROW)
    zeros_acc = jnp.zeros((acc_rows, hid), jnp.float32)
    zeros_cnt = jnp.zeros((cacc_rows, 16), jnp.float32)
    ones_row = jnp.ones((ROW, 16), jnp.float32)

    gidx, cidx = pl.pallas_call(
        functools.partial(_prep_body, n_nodes=n),
        out_shape=[jax.ShapeDtypeStruct((rows_pad, ROW), jnp.int32),
                   jax.ShapeDtypeStruct((rows_pad, ROW), jnp.int32)],
    )(etg_p, src_p, etc_p, dstc_p)

    cnt_pass = _make_cnt_pass(cacc_rows, rows_pad)
    cnt_part = cnt_pass(cidx, ones_row, zeros_cnt)

    H0 = pl.pallas_call(
        _proj_body,
        out_shape=jax.ShapeDtypeStruct((n_ch, n, hid), jnp.float32),
    )(x, W_gcn)

    edge_pass = _make_edge_pass(acc_rows, hid, rows_pad)

    scale_calls = [
        pl.pallas_call(
            _make_scale_body(c, n_rel, n),
            out_shape=jax.ShapeDtypeStruct((n_rel * n, hid), jnp.float32),
        ) for c in range(n_ch)
    ]

    BLK = 2000
    assert n % BLK == 0 and n_rel == 3
    nb = n // BLK
    grid = (nb,)
    combine_calls = []
    for c in range(n_ch):
        combine_calls.append(pl.pallas_call(
            _make_combine_body(c, n_rel),
            grid=grid,
            in_specs=[
                pl.BlockSpec((NC, BLK, hid), lambda i: (0, i, 0)),
                pl.BlockSpec((NC, BLK, 16), lambda i: (0, i, 0)),
                pl.BlockSpec((NC, BLK, 16), lambda i, _nb=nb: (0, _nb + i, 0)),
                pl.BlockSpec((NC, BLK, 16),
                             lambda i, _nb=nb: (0, 2 * _nb + i, 0)),
                pl.BlockSpec((BLK, hid), lambda i: (i, 0)),
                pl.BlockSpec((n_ch, n_rel_full), lambda i: (0, 0)),
                pl.BlockSpec((hid, hid), lambda i: (0, 0)),
                pl.BlockSpec((1, hid), lambda i: (0, 0)),
            ],
            out_specs=pl.BlockSpec((BLK, hid), lambda i: (i, 0)),
            out_shape=jax.ShapeDtypeStruct((n, hid), jnp.float32),
        ))

    H = [H0[c] for c in range(n_ch)]
    for l in range(n_layer):
        newH = []
        for c in range(n_ch):
            h3 = scale_calls[c](H[c], gt_weight[l])
            part = edge_pass(h3, gidx, dsts_p, zeros_acc)
            newH.append(combine_calls[c](
                part, cnt_part, cnt_part, cnt_part, H[c], gt_weight[l],
                Wg, bg.reshape(1, hid)))
        H = newH

    y = pl.pallas_call(
        _head_body,
        out_shape=jax.ShapeDtypeStruct((n, num_class), jnp.float32),
    )(H[0], H[1], W1, b1.reshape(1, hid), W2, b2.reshape(1, num_class))
    return y
